# baseline (speedup 1.0000x reference)
import functools

import jax
import jax.numpy as jnp
from jax import lax
from jax.experimental import pallas as pl
from jax.experimental.pallas import tpu as pltpu
from jax.experimental.pallas import tpu_sc as plsc

VOCAB = 100000
EMB = 32
HID = 128
B = 1024
CTX = 50

_NC = 2
_NS = 16
_NW = _NC * _NS
_B_PER_W = B // _NW
_IDX_PER_W = _B_PER_W * CTX

V_TILE = 2048


def _gather_sum_sc(idx_flat, emb):
    mesh = plsc.VectorSubcoreMesh(core_axis_name="c", subcore_axis_name="s")

    @functools.partial(
        pl.kernel,
        mesh=mesh,
        out_type=jax.ShapeDtypeStruct((B, EMB), jnp.float32),
        scratch_types=[
            pltpu.VMEM((_IDX_PER_W,), jnp.int32),
            pltpu.VMEM((_IDX_PER_W, EMB), jnp.float32),
            pltpu.VMEM((_B_PER_W, EMB), jnp.float32),
            pltpu.SemaphoreType.DMA,
        ],
        compiler_params=pltpu.CompilerParams(use_tc_tiling_on_sc=False),
    )
    def gather_sum(idx_hbm, table_hbm, out_hbm, idx_v, rows_v, acc_v, sem):
        wid = lax.axis_index("s") * _NC + lax.axis_index("c")
        base = wid * _IDX_PER_W
        pltpu.sync_copy(idx_hbm.at[pl.ds(base, _IDX_PER_W)], idx_v)
        pltpu.async_copy(table_hbm.at[idx_v], rows_v, sem).wait()

        def batch_body(b, carry):
            r0 = jnp.zeros((16,), jnp.float32)
            r1 = jnp.zeros((16,), jnp.float32)
            row = b * CTX
            for c in range(CTX):
                r0 = r0 + rows_v[row + c, pl.ds(0, 16)]
                r1 = r1 + rows_v[row + c, pl.ds(16, 16)]
            acc_v[b, pl.ds(0, 16)] = r0
            acc_v[b, pl.ds(16, 16)] = r1
            return carry

        lax.fori_loop(0, _B_PER_W, batch_body, 0)
        pltpu.sync_copy(acc_v, out_hbm.at[pl.ds(wid * _B_PER_W, _B_PER_W)])

    return gather_sum(idx_flat, emb)


def _mlp_kernel(x_ref, w1_ref, b1_ref, w2_ref, b2_ref, out_ref, h_ref):
    @pl.when(pl.program_id(0) == 0)
    def _():
        h = lax.dot_general(
            x_ref[...], w1_ref[...], (((1,), (1,)), ((), ())),
            preferred_element_type=jnp.float32,
        )
        h_ref[...] = jnp.maximum(h + b1_ref[...], 0.0)

    out_ref[...] = lax.dot_general(
        h_ref[...], w2_ref[...], (((1,), (1,)), ((), ())),
        preferred_element_type=jnp.float32,
    ) + b2_ref[...]


def _mlp_tc(x, W1, b1, W2, b2):
    n_tiles = pl.cdiv(VOCAB, V_TILE)
    return pl.pallas_call(
        _mlp_kernel,
        grid=(n_tiles,),
        in_specs=[
            pl.BlockSpec((B, EMB), lambda i: (0, 0)),
            pl.BlockSpec((HID, EMB), lambda i: (0, 0)),
            pl.BlockSpec((1, HID), lambda i: (0, 0)),
            pl.BlockSpec((V_TILE, HID), lambda i: (i, 0)),
            pl.BlockSpec((1, V_TILE), lambda i: (0, i)),
        ],
        out_specs=pl.BlockSpec((B, V_TILE), lambda i: (0, i)),
        out_shape=jax.ShapeDtypeStruct((B, VOCAB), jnp.float32),
        scratch_shapes=[pltpu.VMEM((B, HID), jnp.float32)],
        compiler_params=pltpu.CompilerParams(
            dimension_semantics=("arbitrary",),
        ),
    )(x, W1, b1.reshape(1, HID), W2, b2.reshape(1, VOCAB))


def kernel(inp, emb, W1, b1, W2, b2):
    idx_flat = inp.reshape(-1).astype(jnp.int32)
    x = _gather_sum_sc(idx_flat, emb)
    return _mlp_tc(x, W1, b1, W2, b2)


# speedup vs baseline: 1.0022x; 1.0022x over previous
import functools

import jax
import jax.numpy as jnp
from jax import lax
from jax.experimental import pallas as pl
from jax.experimental.pallas import tpu as pltpu
from jax.experimental.pallas import tpu_sc as plsc

VOCAB = 100000
EMB = 32
HID = 128
B = 1024
CTX = 50

_NC = 2
_NS = 16
_NW = _NC * _NS
_B_PER_W = B // _NW
_IDX_PER_W = _B_PER_W * CTX

V_TILE = 4096


def _gather_sum_sc(idx_flat, emb):
    mesh = plsc.VectorSubcoreMesh(core_axis_name="c", subcore_axis_name="s")

    @functools.partial(
        pl.kernel,
        mesh=mesh,
        out_type=jax.ShapeDtypeStruct((B, EMB), jnp.float32),
        scratch_types=[
            pltpu.VMEM((_IDX_PER_W,), jnp.int32),
            pltpu.VMEM((_IDX_PER_W, EMB), jnp.float32),
            pltpu.VMEM((_B_PER_W, EMB), jnp.float32),
            pltpu.SemaphoreType.DMA,
        ],
        compiler_params=pltpu.CompilerParams(use_tc_tiling_on_sc=False),
    )
    def gather_sum(idx_hbm, table_hbm, out_hbm, idx_v, rows_v, acc_v, sem):
        wid = lax.axis_index("s") * _NC + lax.axis_index("c")
        base = wid * _IDX_PER_W
        pltpu.sync_copy(idx_hbm.at[pl.ds(base, _IDX_PER_W)], idx_v)
        pltpu.async_copy(table_hbm.at[idx_v], rows_v, sem).wait()

        def batch_body(b, carry):
            r0 = jnp.zeros((16,), jnp.float32)
            r1 = jnp.zeros((16,), jnp.float32)
            row = b * CTX
            for c in range(CTX):
                r0 = r0 + rows_v[row + c, pl.ds(0, 16)]
                r1 = r1 + rows_v[row + c, pl.ds(16, 16)]
            acc_v[b, pl.ds(0, 16)] = r0
            acc_v[b, pl.ds(16, 16)] = r1
            return carry

        lax.fori_loop(0, _B_PER_W, batch_body, 0)
        pltpu.sync_copy(acc_v, out_hbm.at[pl.ds(wid * _B_PER_W, _B_PER_W)])

    return gather_sum(idx_flat, emb)


def _mlp_kernel(x_ref, w1_ref, b1_ref, w2_ref, b2_ref, out_ref, h_ref):
    @pl.when(pl.program_id(0) == 0)
    def _():
        h = lax.dot_general(
            x_ref[...], w1_ref[...], (((1,), (1,)), ((), ())),
            preferred_element_type=jnp.float32,
        )
        h_ref[...] = jnp.maximum(h + b1_ref[...], 0.0)

    out_ref[...] = lax.dot_general(
        h_ref[...], w2_ref[...], (((1,), (1,)), ((), ())),
        preferred_element_type=jnp.float32,
    ) + b2_ref[...]


def _mlp_tc(x, W1, b1, W2, b2):
    n_tiles = pl.cdiv(VOCAB, V_TILE)
    return pl.pallas_call(
        _mlp_kernel,
        grid=(n_tiles,),
        in_specs=[
            pl.BlockSpec((B, EMB), lambda i: (0, 0)),
            pl.BlockSpec((HID, EMB), lambda i: (0, 0)),
            pl.BlockSpec((1, HID), lambda i: (0, 0)),
            pl.BlockSpec((V_TILE, HID), lambda i: (i, 0)),
            pl.BlockSpec((1, V_TILE), lambda i: (0, i)),
        ],
        out_specs=pl.BlockSpec((B, V_TILE), lambda i: (0, i)),
        out_shape=jax.ShapeDtypeStruct((B, VOCAB), jnp.float32),
        scratch_shapes=[pltpu.VMEM((B, HID), jnp.float32)],
        compiler_params=pltpu.CompilerParams(
            dimension_semantics=("arbitrary",),
        ),
    )(x, W1, b1.reshape(1, HID), W2, b2.reshape(1, VOCAB))


def kernel(inp, emb, W1, b1, W2, b2):
    idx_flat = inp.reshape(-1).astype(jnp.int32)
    x = _gather_sum_sc(idx_flat, emb)
    return _mlp_tc(x, W1, b1, W2, b2)
